# baseline (speedup 1.0000x reference)
import functools

import jax
import jax.numpy as jnp
from jax import lax
from jax.experimental import pallas as pl
from jax.experimental.pallas import tpu as pltpu
from jax.experimental.pallas import tpu_sc as plsc

NC, NS = 2, 16
NW = NC * NS
CHUNK = 128
LANES = 16
SUB = 256


def _sc_body(rpw, d, table2, idx, pos, out, idx_v, gidx_v, grows_v,
             pos_v, out_v, gsem, psem):
    wid = lax.axis_index("s") * NC + lax.axis_index("c")
    base = wid * rpw
    seq = pos.shape[0]
    p0 = (wid % (seq // rpw)) * rpw
    nch = rpw // CHUNK
    pltpu.sync_copy(idx.at[wid], idx_v)
    for j in range(nch):
        for g in range(CHUNK // LANES):
            sl = pl.ds(g * LANES, LANES)
            gidx_v[j, sl] = idx_v[j, sl] >> 1
    csub = SUB // CHUNK
    for sb in range(rpw // SUB):
        pcopy = pltpu.async_copy(pos.at[pl.ds(p0 + sb * SUB, SUB)], pos_v, psem)
        gathers = [
            pltpu.async_copy(
                table2.at[gidx_v.at[sb * csub + k]],
                grows_v.at[pl.ds(k * CHUNK, CHUNK)],
                gsem,
            )
            for k in range(csub)
        ]
        for g in gathers:
            g.wait()
        pcopy.wait()

        def sel_add(g, carry):
            i0 = g * LANES
            row = sb * csub + i0 // CHUNK
            tvec = idx_v[row, pl.ds(i0 % CHUNK, LANES)]
            offs = (tvec & 1) * d
            for l in range(LANES):
                i = i0 + l
                off = offs[l]
                for q in range(d // LANES):
                    sl = pl.ds(q * LANES, LANES)
                    out_v[i, sl] = (
                        grows_v[i, pl.ds(off + q * LANES, LANES)] + pos_v[i, sl]
                    )
            return carry

        lax.fori_loop(0, SUB // LANES, sel_add, 0)
        pltpu.sync_copy(out_v, out.at[pl.ds(base + sb * SUB, SUB)])


@jax.jit
def _embed(idx3, table2, pos):
    nw, nch, chunk = idx3.shape
    rpw = nch * chunk
    d = table2.shape[1] // 2
    mesh = plsc.VectorSubcoreMesh(
        core_axis_name="c", subcore_axis_name="s", num_cores=NC, num_subcores=NS
    )
    f = pl.kernel(
        functools.partial(_sc_body, rpw, d),
        out_type=jax.ShapeDtypeStruct((nw * rpw, d), jnp.float32),
        mesh=mesh,
        scratch_types=[
            pltpu.VMEM((nch, chunk), jnp.int32),
            pltpu.VMEM((nch, chunk), jnp.int32),
            pltpu.VMEM((SUB, 2 * d), jnp.float32),
            pltpu.VMEM((SUB, d), jnp.float32),
            pltpu.VMEM((SUB, d), jnp.float32),
            pltpu.SemaphoreType.DMA,
            pltpu.SemaphoreType.DMA,
        ],
        compiler_params=pltpu.CompilerParams(use_tc_tiling_on_sc=True),
    )
    return f(table2, idx3, pos)


def kernel(token_indices, token_table, pos_table):
    b, s = token_indices.shape
    rows = b * s
    rpw = rows // NW
    v, d = token_table.shape
    assert rows % NW == 0 and rpw % CHUNK == 0 and s % rpw == 0
    assert v % 2 == 0 and 2 * d == 128 and rpw % SUB == 0
    table2 = token_table.reshape(v // 2, 2 * d)
    idx3 = token_indices.astype(jnp.int32).reshape(NW, rpw // CHUNK, CHUNK)
    out = _embed(idx3, table2, pos_table)
    return out.reshape(b, s, d)
